# Initial kernel scaffold; baseline (speedup 1.0000x reference)
import jax
import jax.numpy as jnp
from jax.experimental import pallas as pl
from jax.experimental.pallas import tpu as pltpu

_SEQ_BLOCK = 256


def _add_pe_block(x_ref, pe_ref, o_ref):
    o_ref[...] = x_ref[...] + pe_ref[:, None, :]


def kernel(x, pe):
    seq_len, batch, d_model = x.shape
    grid = (seq_len // _SEQ_BLOCK,)
    return pl.pallas_call(
        _add_pe_block,
        grid=grid,
        in_specs=[
            pl.BlockSpec((_SEQ_BLOCK, batch, d_model), lambda g: (g, 0, 0)),
            pl.BlockSpec((_SEQ_BLOCK, d_model), lambda g: (g, 0)),
        ],
        out_specs=pl.BlockSpec((_SEQ_BLOCK, batch, d_model), lambda g: (g, 0, 0)),
        out_shape=jax.ShapeDtypeStruct((seq_len, batch, d_model), x.dtype),
    )(x, pe)


# speedup vs baseline: 4.7000x; 4.7000x over previous
import jax
import jax.numpy as jnp
from jax.experimental import pallas as pl
from jax.experimental.pallas import tpu as pltpu

_SEQ_BLOCK = 256


def _add_pe_block(x_ref, pe_ref, o_ref):
    o_ref[...] = x_ref[...] + pe_ref[...][:, None, :]


def kernel(x, pe):
    seq_len, batch, d_model = x.shape
    grid = (seq_len // _SEQ_BLOCK,)
    return pl.pallas_call(
        _add_pe_block,
        grid=grid,
        in_specs=[
            pl.BlockSpec((_SEQ_BLOCK, batch, d_model), lambda g: (g, 0, 0)),
            pl.BlockSpec((_SEQ_BLOCK, d_model), lambda g: (g, 0)),
        ],
        out_specs=pl.BlockSpec((_SEQ_BLOCK, batch, d_model), lambda g: (g, 0, 0)),
        out_shape=jax.ShapeDtypeStruct((seq_len, batch, d_model), x.dtype),
    )(x, pe)
